# baseline (speedup 1.0000x reference)
import functools

import jax
import jax.numpy as jnp
from jax import lax
from jax.experimental import pallas as pl
from jax.experimental.pallas import tpu as pltpu
from jax.experimental.pallas import tpu_sc as plsc

_B, _S, _D = 4096, 50, 128
_N = _B * _S
_NW = 32
_PER_W = _N // _NW
_CHUNK = 200
_NCH = _PER_W // _CHUNK
_NBUF = 4

_mesh = plsc.VectorSubcoreMesh(core_axis_name="c", subcore_axis_name="s")


@functools.partial(
    pl.kernel,
    mesh=_mesh,
    out_type=jax.ShapeDtypeStruct((_N, _D), jnp.float32),
    scratch_types=[
        pltpu.VMEM((_PER_W,), jnp.int32),
        pltpu.VMEM((_NBUF, _CHUNK, _D), jnp.float32),
        pltpu.SemaphoreType.DMA,
        pltpu.SemaphoreType.DMA,
    ],
    compiler_params=pltpu.CompilerParams(use_tc_tiling_on_sc=True),
)
def _gather_kernel(ids_hbm, table_hbm, out_hbm, idx_v, rows_v, gsem, ssem):
    wid = lax.axis_index("s") * 2 + lax.axis_index("c")
    base = wid * _PER_W
    pltpu.sync_copy(ids_hbm.at[pl.ds(base, _PER_W)], idx_v)

    def gather(c, buf):
        pltpu.async_copy(
            table_hbm.at[idx_v.at[pl.ds(c * _CHUNK, _CHUNK)]],
            rows_v.at[buf], gsem)

    def gwait(buf):
        pltpu.make_async_copy(
            table_hbm.at[pl.ds(0, _CHUNK)], rows_v.at[buf], gsem).wait()

    def scatter(c, buf):
        pltpu.async_copy(
            rows_v.at[buf], out_hbm.at[pl.ds(base + c * _CHUNK, _CHUNK)], ssem)

    def swait(buf):
        pltpu.make_async_copy(
            rows_v.at[buf], out_hbm.at[pl.ds(base, _CHUNK)], ssem).wait()

    for k in range(_NBUF):
        gather(k, k)

    gwait(0)
    scatter(0, 0)
    gwait(1)
    scatter(1, 1)

    swait(0)
    gather(4, 0)
    gwait(2)
    scatter(2, 2)

    swait(1)
    gather(5, 1)
    gwait(3)
    scatter(3, 3)

    def body(i, carry):
        c = 4 * i
        for j in range(4):
            swait(j)
            gather(c + j + 2, (j + 2) % 4)
            gwait(j)
            scatter(c + j, j)
        return carry

    lax.fori_loop(1, (_NCH - 2) // 4, body, 0)

    swait(2)
    gather(_NCH - 2, 2)
    gwait(0)
    scatter(_NCH - 4, 0)

    swait(3)
    gather(_NCH - 1, 3)
    gwait(1)
    scatter(_NCH - 3, 1)

    swait(0)
    gwait(2)
    scatter(_NCH - 2, 2)

    swait(1)
    gwait(3)
    scatter(_NCH - 1, 3)

    swait(2)
    swait(3)


def kernel(word_ids, table):
    ids_t = word_ids.T.reshape(-1).astype(jnp.int32)
    out = _gather_kernel(ids_t, table)
    return jnp.transpose(out.reshape(_S, _B, _D), (1, 0, 2))


# speedup vs baseline: 1.0099x; 1.0099x over previous
import functools

import jax
import jax.numpy as jnp
from jax import lax
from jax.experimental import pallas as pl
from jax.experimental.pallas import tpu as pltpu
from jax.experimental.pallas import tpu_sc as plsc

_B, _S, _D = 4096, 50, 128
_N = _B * _S
_NW = 32
_PER_W = _N // _NW
_CHUNK = 320
_NCH = _PER_W // _CHUNK

_mesh = plsc.VectorSubcoreMesh(core_axis_name="c", subcore_axis_name="s")


@functools.partial(
    pl.kernel,
    mesh=_mesh,
    out_type=jax.ShapeDtypeStruct((_N, _D), jnp.float32),
    scratch_types=[
        pltpu.VMEM((_PER_W,), jnp.int32),
        pltpu.VMEM((3, _CHUNK, _D), jnp.float32),
        pltpu.SemaphoreType.DMA,
        pltpu.SemaphoreType.DMA,
    ],
    compiler_params=pltpu.CompilerParams(use_tc_tiling_on_sc=True),
)
def _gather_kernel(ids_hbm, table_hbm, out_hbm, idx_v, rows_v, gsem, ssem):
    wid = lax.axis_index("s") * 2 + lax.axis_index("c")
    base = wid * _PER_W
    pltpu.sync_copy(ids_hbm.at[pl.ds(base, _PER_W)], idx_v)

    def gather(c, buf):
        pltpu.async_copy(
            table_hbm.at[idx_v.at[pl.ds(c * _CHUNK, _CHUNK)]],
            rows_v.at[buf], gsem)

    def gwait(buf):
        pltpu.make_async_copy(
            table_hbm.at[pl.ds(0, _CHUNK)], rows_v.at[buf], gsem).wait()

    def scatter(c, buf):
        pltpu.async_copy(
            rows_v.at[buf], out_hbm.at[pl.ds(base + c * _CHUNK, _CHUNK)], ssem)

    def swait(buf):
        pltpu.make_async_copy(
            rows_v.at[buf], out_hbm.at[pl.ds(base, _CHUNK)], ssem).wait()

    gather(0, 0)
    gather(1, 1)

    gwait(0)
    scatter(0, 0)
    gather(2, 2)

    gwait(1)
    scatter(1, 1)
    swait(1)
    gather(3, 0)

    gwait(2)
    scatter(2, 2)
    swait(2)
    gather(4, 1)

    def body(i, carry):
        c = 3 * i
        for j in range(3):
            buf = j
            gwait(buf)
            scatter(c + j, buf)
            swait(buf)
            gather(c + j + 2, (j + 2) % 3)
        return carry

    lax.fori_loop(1, (_NCH - 2) // 3, body, 0)

    gwait(0)
    scatter(_NCH - 2, 0)
    swait(0)
    gwait(1)
    scatter(_NCH - 1, 1)
    swait(1)
    swait(2)


def kernel(word_ids, table):
    ids_t = word_ids.T.reshape(-1).astype(jnp.int32)
    out = _gather_kernel(ids_t, table)
    return jnp.transpose(out.reshape(_S, _B, _D), (1, 0, 2))
